# baseline (speedup 1.0000x reference)
import functools

import jax
import jax.numpy as jnp
from jax import lax
from jax.experimental import pallas as pl
from jax.experimental.pallas import tpu as pltpu
from jax.experimental.pallas import tpu_sc as plsc

DIMS = 64
NUMS_M = 1000000
L = 50
LPAD = 64
GB = 2
GW = GB * L
GWP = 104
EMBW = 128
NC, NS = 2, 16
NW = NC * NS
NBUF = 3
LANES = 16
NK = DIMS // LANES


TBLK = 128
NRING = 2


def _sc_table_rm(emb_t, v):
    ntb = v // TBLK
    iters = -(-ntb // NW)
    it2 = -(-iters // NRING) * NRING
    mesh = plsc.VectorSubcoreMesh(
        core_axis_name="c", subcore_axis_name="s",
        num_cores=NC, num_subcores=NS)

    @functools.partial(
        pl.kernel,
        out_type=jax.ShapeDtypeStruct((v, EMBW), jnp.float32),
        mesh=mesh,
        scratch_types=[
            pltpu.VMEM((NRING, DIMS, TBLK), jnp.float32),
            pltpu.VMEM((NRING, TBLK, EMBW), jnp.float32),
            pltpu.SemaphoreType.DMA,
            pltpu.SemaphoreType.DMA,
            pltpu.SemaphoreType.DMA,
            pltpu.SemaphoreType.DMA,
        ],
        compiler_params=pltpu.CompilerParams(use_tc_tiling_on_sc=True,
                                             needs_layout_passes=False),
    )
    def body(embt_hbm, out_hbm, ibuf, obuf, si0, si1, so0, so1):
        isems = (si0, si1)
        osems = (so0, so1)
        wid = lax.axis_index("s") * NC + lax.axis_index("c")
        iota16 = lax.broadcasted_iota(jnp.int32, (LANES,), 0)

        def issue_in(i, b):
            t = i * NW + wid
            @pl.when(t < ntb)
            def _():
                off = pl.multiple_of(t * TBLK, TBLK)
                pltpu.async_copy(
                    embt_hbm.at[:, pl.ds(off, TBLK)], ibuf.at[b],
                    isems[b])

        def wait_in(i, b):
            @pl.when(i * NW + wid < ntb)
            def _():
                pltpu.make_async_copy(
                    embt_hbm.at[:, pl.ds(0, TBLK)], ibuf.at[b],
                    isems[b]).wait()

        def issue_out(i, b):
            t = i * NW + wid
            @pl.when(t < ntb)
            def _():
                off = pl.multiple_of(t * TBLK, TBLK)
                pltpu.async_copy(
                    obuf.at[b], out_hbm.at[pl.ds(off, TBLK)], osems[b])

        def wait_out(i, b):
            @pl.when(jnp.logical_and(i >= 0, i * NW + wid < ntb))
            def _():
                pltpu.make_async_copy(
                    obuf.at[b], out_hbm.at[pl.ds(0, TBLK)],
                    osems[b]).wait()

        for b in range(NRING):
            issue_in(b, b)

        def step(c, carry):
            for b in range(NRING):
                i = c * NRING + b
                wait_in(i, b)
                wait_out(i - NRING, b)

                @pl.when(i * NW + wid < ntb)
                def _():
                    src = ibuf.at[b]
                    for r in range(TBLK):
                        cvec = jnp.full((LANES,), r, jnp.int32)
                        for k in range(NK):
                            g = plsc.load_gather(
                                src, [k * LANES + iota16, cvec])
                            obuf[b, r, pl.ds(k * LANES, LANES)] = g

                issue_out(i, b)
                issue_in(i + NRING, b)
            return carry

        lax.fori_loop(0, it2 // NRING, step, 0)
        for b in range(NRING):
            wait_out(it2 - NRING + b, b)

    return body(emb_t)


def _sc_bag_sum(gidx, embp, nrows):
    gpw = nrows // GB // NW
    rpw = nrows // NW
    mesh = plsc.VectorSubcoreMesh(
        core_axis_name="c", subcore_axis_name="s",
        num_cores=NC, num_subcores=NS)

    @functools.partial(
        pl.kernel,
        out_type=jax.ShapeDtypeStruct((nrows * DIMS,), jnp.float32),
        mesh=mesh,
        scratch_types=[
            pltpu.VMEM((gpw * GWP,), jnp.int32),
            pltpu.VMEM((NBUF, GWP, EMBW), jnp.float32),
            pltpu.VMEM((rpw * DIMS,), jnp.float32),
            pltpu.SemaphoreType.DMA,
            pltpu.SemaphoreType.DMA,
            pltpu.SemaphoreType.DMA,
        ],
        compiler_params=pltpu.CompilerParams(use_tc_tiling_on_sc=True),
    )
    def body(gidx_hbm, emb_hbm, out_hbm, gidx_v, bufs, out_v, s0, s1, s2):
        sems = (s0, s1, s2)
        wid = lax.axis_index("s") * NC + lax.axis_index("c")
        gbase = wid * gpw
        pltpu.sync_copy(gidx_hbm.at[pl.ds(gbase * GWP, gpw * GWP)], gidx_v)

        def issue(g, b):
            off = pl.multiple_of(g * GWP, 8)
            pltpu.async_copy(
                emb_hbm.at[gidx_v.at[pl.ds(off, GWP)]], bufs.at[b], sems[b])

        def drain(b):
            pltpu.make_async_copy(
                emb_hbm.at[gidx_v.at[pl.ds(0, GWP)]], bufs.at[b],
                sems[b]).wait()

        for b in range(NBUF):
            issue(b, b)

        def step(c, carry):
            g0 = c * NBUF
            for b in range(NBUF):
                g = g0 + b
                drain(b)
                obase = g * (GB * DIMS)
                for bag in range(GB):
                    acc = [None] * NK
                    for j in range(L):
                        row = bag * L + j
                        for k in range(NK):
                            v = bufs[b, row, pl.ds(k * LANES, LANES)]
                            acc[k] = v if acc[k] is None else acc[k] + v
                    for k in range(NK):
                        out_v[pl.ds(obase + bag * DIMS + k * LANES,
                                    LANES)] = acc[k]
                nxt = g + NBUF
                @pl.when(nxt < gpw)
                def _():
                    issue(nxt, b)
            return carry

        lax.fori_loop(0, gpw // NBUF, step, 0)
        pltpu.sync_copy(out_v, out_hbm.at[pl.ds(wid * rpw * DIMS,
                                                rpw * DIMS)])

    return body(gidx, embp)


def _tc_head(sum_q, sum_d, m_q, m_d, tdiff, idx_q, idx_d, emb0,
             qw, qb, dw, db):
    bq = sum_q.shape[0]
    bd = sum_d.shape[0]
    h = qw.shape[0]
    qblk = 512
    nqb = bq // qblk
    dch = 1024
    ndch = bd // dch

    def pool_tower(s, m, td, idx, e0, w_ref, b_ref):
        cnt = jnp.sum(jnp.where(idx > 0, 1.0, 0.0), axis=1, keepdims=True)
        s = s + jnp.dot(m, td, preferred_element_type=jnp.float32)
        x = (s - (jnp.float32(L) - cnt) * e0) / cnt
        y = jnp.dot(x, w_ref[...].T, preferred_element_type=jnp.float32)
        y = jnp.maximum(y + b_ref[...], 0.0)
        n = jnp.sqrt(jnp.sum(y * y, axis=1, keepdims=True))
        return y / jnp.maximum(n, 1e-12)

    def body(sq_ref, sd_ref, mq_ref, md_ref, td_ref, iq_ref, id_ref,
             e0_ref, qw_ref, qb_ref, dw_ref, db_ref, out_ref, dn_ref):
        i = pl.program_id(0)

        @pl.when(i == 0)
        def _():
            dn_ref[...] = pool_tower(sd_ref[...], md_ref[...], td_ref[...],
                                     id_ref[...], e0_ref[...],
                                     dw_ref, db_ref)
            out_ref[...] = jnp.zeros((1, 1), jnp.float32)

        qn = pool_tower(sq_ref[...], mq_ref[...], td_ref[...], iq_ref[...],
                        e0_ref[...], qw_ref, qb_ref)

        def chunk(c, carry):
            sums, diag = carry
            dchunk = dn_ref[pl.ds(c * dch, dch), :]
            s = jnp.dot(qn, dchunk.T, preferred_element_type=jnp.float32)
            sums = sums + jnp.sum(jnp.exp(s), axis=1, keepdims=True)
            rows = lax.broadcasted_iota(jnp.int32, (qblk, dch), 0) + i * qblk
            cols = lax.broadcasted_iota(jnp.int32, (qblk, dch), 1) + c * dch
            diag = diag + jnp.sum(jnp.where(rows == cols, s, 0.0),
                                  axis=1, keepdims=True)
            return sums, diag

        z = jnp.zeros((qblk, 1), jnp.float32)
        sums, diag = lax.fori_loop(0, ndch, chunk, (z, z))
        out_ref[...] += (jnp.sum(jnp.log(sums) - diag) / bq).reshape(1, 1)

    out = pl.pallas_call(
        body,
        grid=(nqb,),
        in_specs=[
            pl.BlockSpec((qblk, DIMS), lambda i: (i, 0)),
            pl.BlockSpec((bd, DIMS), lambda i: (0, 0)),
            pl.BlockSpec((qblk, TBLK // 2), lambda i: (i, 0)),
            pl.BlockSpec((bd, TBLK // 2), lambda i: (0, 0)),
            pl.BlockSpec((TBLK // 2, DIMS), lambda i: (0, 0)),
            pl.BlockSpec((qblk, LPAD), lambda i: (i, 0)),
            pl.BlockSpec((bd, LPAD), lambda i: (0, 0)),
            pl.BlockSpec((1, DIMS), lambda i: (0, 0)),
            pl.BlockSpec((h, DIMS), lambda i: (0, 0)),
            pl.BlockSpec((1, h), lambda i: (0, 0)),
            pl.BlockSpec((h, DIMS), lambda i: (0, 0)),
            pl.BlockSpec((1, h), lambda i: (0, 0)),
        ],
        out_specs=pl.BlockSpec((1, 1), lambda i: (0, 0)),
        out_shape=jax.ShapeDtypeStruct((1, 1), jnp.float32),
        scratch_shapes=[pltpu.VMEM((bd, h), jnp.float32)],
    )(sum_q, sum_d, m_q, m_d, tdiff, idx_q, idx_d, emb0, qw,
      qb.reshape(1, h), dw, db.reshape(1, h))
    return out[0, 0]


def kernel(query, doc, negs, emb, qd1_w, qd1_b, dd1_w, dd1_b):
    b = query.shape[0]
    v = emb.shape[0]
    t0 = (v // TBLK) * TBLK
    tn = v - t0
    idx = jnp.concatenate([query, doc, negs], axis=0)
    nrows = idx.shape[0]
    idxp = jnp.pad(idx, ((0, 0), (0, LPAD - L)))
    idx_rm = jnp.where(idx >= t0, idx - t0, idx)
    m = jnp.sum((idx[:, :, None] == (t0 + jnp.arange(tn))[None, None, :])
                .astype(jnp.float32), axis=1)
    tdiff = emb[t0:] - emb[:tn]
    grp = jnp.pad(idx_rm.reshape(nrows // GB, GW), ((0, 0), (0, GWP - GW)))
    ng = nrows // GB
    spread = (lax.broadcasted_iota(jnp.int32, (ng, GWP), 0) * 997
              + lax.broadcasted_iota(jnp.int32, (ng, GWP), 1) * 131) % NUMS_M
    col = lax.broadcasted_iota(jnp.int32, (ng, GWP), 1)
    grp = jnp.where(col < GW, grp, spread)
    gidx = grp.reshape(-1)
    table2 = _sc_table_rm(emb.T, v)
    sums = _sc_bag_sum(gidx, table2, nrows).reshape(nrows, DIMS)
    return _tc_head(sums[:b], sums[b:], m[:b], m[b:], tdiff,
                    idxp[:b], idxp[b:], emb[0:1],
                    qd1_w, qd1_b, dd1_w, dd1_b)


# speedup vs baseline: 1.3822x; 1.3822x over previous
import functools

import jax
import jax.numpy as jnp
from jax import lax
from jax.experimental import pallas as pl
from jax.experimental.pallas import tpu as pltpu
from jax.experimental.pallas import tpu_sc as plsc

DIMS = 64
NUMS_M = 1000000
L = 50
LPAD = 64
GB = 2
GW = GB * L
GWP = 104
EMBW = 128
NC, NS = 2, 16
NW = NC * NS
NBUF = 3
LANES = 16
NK = DIMS // LANES


TBLK = 128
NRING = 2


def _sc_table_rm(emb_t, v):
    ntb = v // TBLK
    iters = -(-ntb // NW)
    it2 = -(-iters // NRING) * NRING
    mesh = plsc.VectorSubcoreMesh(
        core_axis_name="c", subcore_axis_name="s",
        num_cores=NC, num_subcores=NS)

    @functools.partial(
        pl.kernel,
        out_type=jax.ShapeDtypeStruct((v, EMBW), jnp.float32),
        mesh=mesh,
        scratch_types=[
            pltpu.VMEM((NRING, DIMS, TBLK), jnp.float32),
            pltpu.VMEM((NRING, TBLK, EMBW), jnp.float32),
            pltpu.SemaphoreType.DMA,
            pltpu.SemaphoreType.DMA,
            pltpu.SemaphoreType.DMA,
            pltpu.SemaphoreType.DMA,
        ],
        compiler_params=pltpu.CompilerParams(use_tc_tiling_on_sc=True,
                                             needs_layout_passes=False),
    )
    def body(embt_hbm, out_hbm, ibuf, obuf, si0, si1, so0, so1):
        isems = (si0, si1)
        osems = (so0, so1)
        wid = lax.axis_index("s") * NC + lax.axis_index("c")
        iota16 = lax.broadcasted_iota(jnp.int32, (LANES,), 0)

        def issue_in(i, b):
            t = i * NW + wid
            @pl.when(t < ntb)
            def _():
                off = pl.multiple_of(t * TBLK, TBLK)
                pltpu.async_copy(
                    embt_hbm.at[:, pl.ds(off, TBLK)], ibuf.at[b],
                    isems[b])

        def wait_in(i, b):
            @pl.when(i * NW + wid < ntb)
            def _():
                pltpu.make_async_copy(
                    embt_hbm.at[:, pl.ds(0, TBLK)], ibuf.at[b],
                    isems[b]).wait()

        def issue_out(i, b):
            t = i * NW + wid
            @pl.when(t < ntb)
            def _():
                off = pl.multiple_of(t * TBLK, TBLK)
                pltpu.async_copy(
                    obuf.at[b], out_hbm.at[pl.ds(off, TBLK)], osems[b])

        def wait_out(i, b):
            @pl.when(jnp.logical_and(i >= 0, i * NW + wid < ntb))
            def _():
                pltpu.make_async_copy(
                    obuf.at[b], out_hbm.at[pl.ds(0, TBLK)],
                    osems[b]).wait()

        for b in range(NRING):
            issue_in(b, b)

        def step(c, carry):
            for b in range(NRING):
                i = c * NRING + b
                wait_in(i, b)
                wait_out(i - NRING, b)

                @pl.when(i * NW + wid < ntb)
                def _():
                    src = ibuf.at[b]

                    @plsc.parallel_loop(0, TBLK, unroll=8)
                    def _(r):
                        cvec = jnp.full((LANES,), 0, jnp.int32) + r
                        gs = [plsc.load_gather(src, [k * LANES + iota16,
                                                     cvec])
                              for k in range(NK)]
                        for k in range(NK):
                            obuf[b, r, pl.ds(k * LANES, LANES)] = gs[k]

                issue_out(i, b)
                issue_in(i + NRING, b)
            return carry

        lax.fori_loop(0, it2 // NRING, step, 0)
        for b in range(NRING):
            wait_out(it2 - NRING + b, b)

    return body(emb_t)


def _sc_bag_sum(gidx, embp, nrows):
    gpw = nrows // GB // NW
    rpw = nrows // NW
    mesh = plsc.VectorSubcoreMesh(
        core_axis_name="c", subcore_axis_name="s",
        num_cores=NC, num_subcores=NS)

    @functools.partial(
        pl.kernel,
        out_type=jax.ShapeDtypeStruct((nrows * DIMS,), jnp.float32),
        mesh=mesh,
        scratch_types=[
            pltpu.VMEM((gpw * GWP,), jnp.int32),
            pltpu.VMEM((NBUF, GWP, EMBW), jnp.float32),
            pltpu.VMEM((rpw * DIMS,), jnp.float32),
            pltpu.SemaphoreType.DMA,
            pltpu.SemaphoreType.DMA,
            pltpu.SemaphoreType.DMA,
        ],
        compiler_params=pltpu.CompilerParams(use_tc_tiling_on_sc=True),
    )
    def body(gidx_hbm, emb_hbm, out_hbm, gidx_v, bufs, out_v, s0, s1, s2):
        sems = (s0, s1, s2)
        wid = lax.axis_index("s") * NC + lax.axis_index("c")
        gbase = wid * gpw
        pltpu.sync_copy(gidx_hbm.at[pl.ds(gbase * GWP, gpw * GWP)], gidx_v)

        def issue(g, b):
            off = pl.multiple_of(g * GWP, 8)
            pltpu.async_copy(
                emb_hbm.at[gidx_v.at[pl.ds(off, GWP)]], bufs.at[b], sems[b])

        def drain(b):
            pltpu.make_async_copy(
                emb_hbm.at[gidx_v.at[pl.ds(0, GWP)]], bufs.at[b],
                sems[b]).wait()

        for b in range(NBUF):
            issue(b, b)

        def step(c, carry):
            g0 = c * NBUF
            for b in range(NBUF):
                g = g0 + b
                drain(b)
                obase = g * (GB * DIMS)
                for bag in range(GB):
                    acc = [None] * NK
                    for j in range(L):
                        row = bag * L + j
                        for k in range(NK):
                            v = bufs[b, row, pl.ds(k * LANES, LANES)]
                            acc[k] = v if acc[k] is None else acc[k] + v
                    for k in range(NK):
                        out_v[pl.ds(obase + bag * DIMS + k * LANES,
                                    LANES)] = acc[k]
                nxt = g + NBUF
                @pl.when(nxt < gpw)
                def _():
                    issue(nxt, b)
            return carry

        lax.fori_loop(0, gpw // NBUF, step, 0)
        pltpu.sync_copy(out_v, out_hbm.at[pl.ds(wid * rpw * DIMS,
                                                rpw * DIMS)])

    return body(gidx, embp)


def _tc_head(sum_q, sum_d, m_q, m_d, tdiff, idx_q, idx_d, emb0,
             qw, qb, dw, db):
    bq = sum_q.shape[0]
    bd = sum_d.shape[0]
    h = qw.shape[0]
    qblk = 512
    nqb = bq // qblk
    dch = 1024
    ndch = bd // dch

    def pool_tower(s, m, td, idx, e0, w_ref, b_ref):
        cnt = jnp.sum(jnp.where(idx > 0, 1.0, 0.0), axis=1, keepdims=True)
        s = s + jnp.dot(m, td, preferred_element_type=jnp.float32)
        x = (s - (jnp.float32(L) - cnt) * e0) / cnt
        y = jnp.dot(x, w_ref[...].T, preferred_element_type=jnp.float32)
        y = jnp.maximum(y + b_ref[...], 0.0)
        n = jnp.sqrt(jnp.sum(y * y, axis=1, keepdims=True))
        return y / jnp.maximum(n, 1e-12)

    def body(sq_ref, sd_ref, mq_ref, md_ref, td_ref, iq_ref, id_ref,
             e0_ref, qw_ref, qb_ref, dw_ref, db_ref, out_ref, dn_ref):
        i = pl.program_id(0)

        @pl.when(i == 0)
        def _():
            dn_ref[...] = pool_tower(sd_ref[...], md_ref[...], td_ref[...],
                                     id_ref[...], e0_ref[...],
                                     dw_ref, db_ref)
            out_ref[...] = jnp.zeros((1, 1), jnp.float32)

        qn = pool_tower(sq_ref[...], mq_ref[...], td_ref[...], iq_ref[...],
                        e0_ref[...], qw_ref, qb_ref)

        def chunk(c, carry):
            sums, diag = carry
            dchunk = dn_ref[pl.ds(c * dch, dch), :]
            s = jnp.dot(qn, dchunk.T, preferred_element_type=jnp.float32)
            sums = sums + jnp.sum(jnp.exp(s), axis=1, keepdims=True)
            rows = lax.broadcasted_iota(jnp.int32, (qblk, dch), 0) + i * qblk
            cols = lax.broadcasted_iota(jnp.int32, (qblk, dch), 1) + c * dch
            diag = diag + jnp.sum(jnp.where(rows == cols, s, 0.0),
                                  axis=1, keepdims=True)
            return sums, diag

        z = jnp.zeros((qblk, 1), jnp.float32)
        sums, diag = lax.fori_loop(0, ndch, chunk, (z, z))
        out_ref[...] += (jnp.sum(jnp.log(sums) - diag) / bq).reshape(1, 1)

    out = pl.pallas_call(
        body,
        grid=(nqb,),
        in_specs=[
            pl.BlockSpec((qblk, DIMS), lambda i: (i, 0)),
            pl.BlockSpec((bd, DIMS), lambda i: (0, 0)),
            pl.BlockSpec((qblk, TBLK // 2), lambda i: (i, 0)),
            pl.BlockSpec((bd, TBLK // 2), lambda i: (0, 0)),
            pl.BlockSpec((TBLK // 2, DIMS), lambda i: (0, 0)),
            pl.BlockSpec((qblk, LPAD), lambda i: (i, 0)),
            pl.BlockSpec((bd, LPAD), lambda i: (0, 0)),
            pl.BlockSpec((1, DIMS), lambda i: (0, 0)),
            pl.BlockSpec((h, DIMS), lambda i: (0, 0)),
            pl.BlockSpec((1, h), lambda i: (0, 0)),
            pl.BlockSpec((h, DIMS), lambda i: (0, 0)),
            pl.BlockSpec((1, h), lambda i: (0, 0)),
        ],
        out_specs=pl.BlockSpec((1, 1), lambda i: (0, 0)),
        out_shape=jax.ShapeDtypeStruct((1, 1), jnp.float32),
        scratch_shapes=[pltpu.VMEM((bd, h), jnp.float32)],
    )(sum_q, sum_d, m_q, m_d, tdiff, idx_q, idx_d, emb0, qw,
      qb.reshape(1, h), dw, db.reshape(1, h))
    return out[0, 0]


def kernel(query, doc, negs, emb, qd1_w, qd1_b, dd1_w, dd1_b):
    b = query.shape[0]
    v = emb.shape[0]
    t0 = (v // TBLK) * TBLK
    tn = v - t0
    idx = jnp.concatenate([query, doc, negs], axis=0)
    nrows = idx.shape[0]
    idxp = jnp.pad(idx, ((0, 0), (0, LPAD - L)))
    idx_rm = jnp.where(idx >= t0, idx - t0, idx)
    m = jnp.sum((idx[:, :, None] == (t0 + jnp.arange(tn))[None, None, :])
                .astype(jnp.float32), axis=1)
    tdiff = emb[t0:] - emb[:tn]
    grp = jnp.pad(idx_rm.reshape(nrows // GB, GW), ((0, 0), (0, GWP - GW)))
    ng = nrows // GB
    spread = (lax.broadcasted_iota(jnp.int32, (ng, GWP), 0) * 997
              + lax.broadcasted_iota(jnp.int32, (ng, GWP), 1) * 131) % NUMS_M
    col = lax.broadcasted_iota(jnp.int32, (ng, GWP), 1)
    grp = jnp.where(col < GW, grp, spread)
    gidx = grp.reshape(-1)
    table2 = _sc_table_rm(emb.T, v)
    sums = _sc_bag_sum(gidx, table2, nrows).reshape(nrows, DIMS)
    return _tc_head(sums[:b], sums[b:], m[:b], m[b:], tdiff,
                    idxp[:b], idxp[b:], emb[0:1],
                    qd1_w, qd1_b, dd1_w, dd1_b)


# speedup vs baseline: 2.2530x; 1.6300x over previous
import functools

import jax
import jax.numpy as jnp
from jax import lax
from jax.experimental import pallas as pl
from jax.experimental.pallas import tpu as pltpu
from jax.experimental.pallas import tpu_sc as plsc

DIMS = 64
NUMS_M = 1000000
L = 50
LPAD = 64
GB = 2
GW = GB * L
GWP = 104
EMBW = 128
NC, NS = 2, 16
NW = NC * NS
NBUF = 3
LANES = 16
NK = DIMS // LANES


def _sc_bag_sum(gidx, embp, nrows):
    gpw = nrows // GB // NW
    rpw = nrows // NW
    mesh = plsc.VectorSubcoreMesh(
        core_axis_name="c", subcore_axis_name="s",
        num_cores=NC, num_subcores=NS)

    @functools.partial(
        pl.kernel,
        out_type=jax.ShapeDtypeStruct((nrows * DIMS,), jnp.float32),
        mesh=mesh,
        scratch_types=[
            pltpu.VMEM((gpw * GWP,), jnp.int32),
            pltpu.VMEM((NBUF, GWP, EMBW), jnp.float32),
            pltpu.VMEM((rpw * DIMS,), jnp.float32),
            pltpu.SemaphoreType.DMA,
            pltpu.SemaphoreType.DMA,
            pltpu.SemaphoreType.DMA,
        ],
        compiler_params=pltpu.CompilerParams(use_tc_tiling_on_sc=False),
    )
    def body(gidx_hbm, emb_hbm, out_hbm, gidx_v, bufs, out_v, s0, s1, s2):
        sems = (s0, s1, s2)
        wid = lax.axis_index("s") * NC + lax.axis_index("c")
        gbase = wid * gpw
        pltpu.sync_copy(gidx_hbm.at[pl.ds(gbase * GWP, gpw * GWP)], gidx_v)

        def issue(g, b):
            off = pl.multiple_of(g * GWP, 8)
            pltpu.async_copy(
                emb_hbm.at[gidx_v.at[pl.ds(off, GWP)]], bufs.at[b], sems[b])

        def drain(b):
            pltpu.make_async_copy(
                emb_hbm.at[gidx_v.at[pl.ds(0, GWP)]], bufs.at[b],
                sems[b]).wait()

        for b in range(NBUF):
            issue(b, b)

        def step(c, carry):
            g0 = c * NBUF
            for b in range(NBUF):
                g = g0 + b
                drain(b)
                obase = g * (GB * DIMS)
                for bag in range(GB):
                    acc = [None] * NK
                    for j in range(L):
                        row = bag * L + j
                        for k in range(NK):
                            v = bufs[b, row, pl.ds(k * LANES, LANES)]
                            acc[k] = v if acc[k] is None else acc[k] + v
                    for k in range(NK):
                        out_v[pl.ds(obase + bag * DIMS + k * LANES,
                                    LANES)] = acc[k]
                nxt = g + NBUF
                @pl.when(nxt < gpw)
                def _():
                    issue(nxt, b)
            return carry

        lax.fori_loop(0, gpw // NBUF, step, 0)
        pltpu.sync_copy(out_v, out_hbm.at[pl.ds(wid * rpw * DIMS,
                                                rpw * DIMS)])

    return body(gidx, embp)


def _tc_head(sum_q, sum_d, idx_q, idx_d, emb0, qw, qb, dw, db):
    bq = sum_q.shape[0]
    bd = sum_d.shape[0]
    h = qw.shape[0]
    qblk = 512
    nqb = bq // qblk
    dch = 1024
    ndch = bd // dch

    def pool_tower(s, idx, e0, w_ref, b_ref):
        cnt = jnp.sum(jnp.where(idx > 0, 1.0, 0.0), axis=1, keepdims=True)
        x = (s - (jnp.float32(L) - cnt) * e0) / cnt
        y = jnp.dot(x, w_ref[...].T, preferred_element_type=jnp.float32)
        y = jnp.maximum(y + b_ref[...], 0.0)
        n = jnp.sqrt(jnp.sum(y * y, axis=1, keepdims=True))
        return y / jnp.maximum(n, 1e-12)

    def body(sq_ref, sd_ref, iq_ref, id_ref, e0_ref, qw_ref, qb_ref,
             dw_ref, db_ref, out_ref, dn_ref):
        i = pl.program_id(0)

        @pl.when(i == 0)
        def _():
            dn_ref[...] = pool_tower(sd_ref[...], id_ref[...], e0_ref[...],
                                     dw_ref, db_ref)
            out_ref[...] = jnp.zeros((1, 1), jnp.float32)

        qn = pool_tower(sq_ref[...], iq_ref[...], e0_ref[...], qw_ref, qb_ref)

        def chunk(c, carry):
            sums, diag = carry
            dchunk = dn_ref[pl.ds(c * dch, dch), :]
            s = jnp.dot(qn, dchunk.T, preferred_element_type=jnp.float32)
            sums = sums + jnp.sum(jnp.exp(s), axis=1, keepdims=True)
            rows = lax.broadcasted_iota(jnp.int32, (qblk, dch), 0) + i * qblk
            cols = lax.broadcasted_iota(jnp.int32, (qblk, dch), 1) + c * dch
            diag = diag + jnp.sum(jnp.where(rows == cols, s, 0.0),
                                  axis=1, keepdims=True)
            return sums, diag

        z = jnp.zeros((qblk, 1), jnp.float32)
        sums, diag = lax.fori_loop(0, ndch, chunk, (z, z))
        out_ref[...] += (jnp.sum(jnp.log(sums) - diag) / bq).reshape(1, 1)

    out = pl.pallas_call(
        body,
        grid=(nqb,),
        in_specs=[
            pl.BlockSpec((qblk, DIMS), lambda i: (i, 0)),
            pl.BlockSpec((bd, DIMS), lambda i: (0, 0)),
            pl.BlockSpec((qblk, LPAD), lambda i: (i, 0)),
            pl.BlockSpec((bd, LPAD), lambda i: (0, 0)),
            pl.BlockSpec((1, DIMS), lambda i: (0, 0)),
            pl.BlockSpec((h, DIMS), lambda i: (0, 0)),
            pl.BlockSpec((1, h), lambda i: (0, 0)),
            pl.BlockSpec((h, DIMS), lambda i: (0, 0)),
            pl.BlockSpec((1, h), lambda i: (0, 0)),
        ],
        out_specs=pl.BlockSpec((1, 1), lambda i: (0, 0)),
        out_shape=jax.ShapeDtypeStruct((1, 1), jnp.float32),
        scratch_shapes=[pltpu.VMEM((bd, h), jnp.float32)],
    )(sum_q, sum_d, idx_q, idx_d, emb0, qw, qb.reshape(1, h), dw,
      db.reshape(1, h))
    return out[0, 0]


def kernel(query, doc, negs, emb, qd1_w, qd1_b, dd1_w, dd1_b):
    b = query.shape[0]
    idx = jnp.concatenate([query, doc, negs], axis=0)
    nrows = idx.shape[0]
    idxp = jnp.pad(idx, ((0, 0), (0, LPAD - L)))
    grp = jnp.pad(idx.reshape(nrows // GB, GW), ((0, 0), (0, GWP - GW)))
    ng = nrows // GB
    spread = (lax.broadcasted_iota(jnp.int32, (ng, GWP), 0) * 997
              + lax.broadcasted_iota(jnp.int32, (ng, GWP), 1) * 131) % NUMS_M
    col = lax.broadcasted_iota(jnp.int32, (ng, GWP), 1)
    grp = jnp.where(col < GW, grp, spread)
    gidx = grp.reshape(-1)
    embp = jnp.pad(emb, ((0, 0), (0, EMBW - DIMS)))
    sums = _sc_bag_sum(gidx, embp, nrows).reshape(nrows, DIMS)
    return _tc_head(sums[:b], sums[b:], idxp[:b], idxp[b:], emb[0:1],
                    qd1_w, qd1_b, dd1_w, dd1_b)


# speedup vs baseline: 2.2989x; 1.0204x over previous
import functools

import jax
import jax.numpy as jnp
from jax import lax
from jax.experimental import pallas as pl
from jax.experimental.pallas import tpu as pltpu
from jax.experimental.pallas import tpu_sc as plsc

DIMS = 64
NUMS_M = 1000000
L = 50
LPAD = 64
GB = 2
GW = GB * L
GWP = 104
EMBW = 128
NC, NS = 2, 16
NW = NC * NS
NBUF = 3
LANES = 16
NK = DIMS // LANES


def _sc_bag_sum(gidx, embp, nrows):
    gpw = nrows // GB // NW
    rpw = nrows // NW
    mesh = plsc.VectorSubcoreMesh(
        core_axis_name="c", subcore_axis_name="s",
        num_cores=NC, num_subcores=NS)

    @functools.partial(
        pl.kernel,
        out_type=jax.ShapeDtypeStruct((nrows * DIMS,), jnp.float32),
        mesh=mesh,
        scratch_types=[
            pltpu.VMEM((gpw * GWP,), jnp.int32),
            pltpu.VMEM((NBUF, GWP, EMBW), jnp.float32),
            pltpu.VMEM((rpw * DIMS,), jnp.float32),
            pltpu.SemaphoreType.DMA,
            pltpu.SemaphoreType.DMA,
            pltpu.SemaphoreType.DMA,
        ],
        compiler_params=pltpu.CompilerParams(use_tc_tiling_on_sc=False),
    )
    def body(gidx_hbm, emb_hbm, out_hbm, gidx_v, bufs, out_v, s0, s1, s2):
        sems = (s0, s1, s2)
        wid = lax.axis_index("s") * NC + lax.axis_index("c")
        gbase = wid * gpw
        pltpu.sync_copy(gidx_hbm.at[pl.ds(gbase * GWP, gpw * GWP)], gidx_v)

        def issue(g, b):
            off = pl.multiple_of(g * GWP, 8)
            pltpu.async_copy(
                emb_hbm.at[gidx_v.at[pl.ds(off, GWP)]], bufs.at[b], sems[b])

        def drain(b):
            pltpu.make_async_copy(
                emb_hbm.at[gidx_v.at[pl.ds(0, GWP)]], bufs.at[b],
                sems[b]).wait()

        for b in range(NBUF):
            issue(b, b)

        def step(c, carry):
            g0 = c * NBUF
            for b in range(NBUF):
                g = g0 + b
                drain(b)
                obase = g * (GB * DIMS)
                for bag in range(GB):
                    acc = [None] * NK
                    for j in range(L):
                        row = bag * L + j
                        for k in range(NK):
                            v = bufs[b, row, pl.ds(k * LANES, LANES)]
                            acc[k] = v if acc[k] is None else acc[k] + v
                    for k in range(NK):
                        out_v[pl.ds(obase + bag * DIMS + k * LANES,
                                    LANES)] = acc[k]
                nxt = g + NBUF
                @pl.when(nxt < gpw)
                def _():
                    issue(nxt, b)
            return carry

        lax.fori_loop(0, gpw // NBUF, step, 0)
        pltpu.sync_copy(out_v, out_hbm.at[pl.ds(wid * rpw * DIMS,
                                                rpw * DIMS)])

    return body(gidx, embp)


def _tc_head(sum_q, sum_d, idx_q, idx_d, emb0, qw, qb, dw, db):
    bq = sum_q.shape[0]
    bd = sum_d.shape[0]
    h = qw.shape[0]
    qblk = 512
    nqb = bq // qblk
    dch = 1024
    ndch = bd // dch

    def pool_tower(s, idx, e0, w_ref, b_ref):
        cnt = jnp.sum(jnp.where(idx > 0, 1.0, 0.0), axis=1, keepdims=True)
        x = (s - (jnp.float32(L) - cnt) * e0) / cnt
        y = jnp.dot(x, w_ref[...].T, preferred_element_type=jnp.float32)
        y = jnp.maximum(y + b_ref[...], 0.0)
        n = jnp.sqrt(jnp.sum(y * y, axis=1, keepdims=True))
        return y / jnp.maximum(n, 1e-12)

    def body(sq_ref, sd_ref, iq_ref, id_ref, e0_ref, qw_ref, qb_ref,
             dw_ref, db_ref, out_ref, dn_ref):
        i = pl.program_id(0)

        @pl.when(i == 0)
        def _():
            dn_ref[...] = pool_tower(sd_ref[...], id_ref[...], e0_ref[...],
                                     dw_ref, db_ref)
            out_ref[...] = jnp.zeros((1, 1), jnp.float32)

        qn = pool_tower(sq_ref[...], iq_ref[...], e0_ref[...], qw_ref, qb_ref)

        s = jnp.dot(qn, dn_ref[...].T, preferred_element_type=jnp.float32)
        sums = jnp.sum(jnp.exp(s), axis=1, keepdims=True)
        rows = lax.broadcasted_iota(jnp.int32, (qblk, bd), 0) + i * qblk
        cols = lax.broadcasted_iota(jnp.int32, (qblk, bd), 1)
        diag = jnp.sum(jnp.where(rows == cols, s, 0.0),
                       axis=1, keepdims=True)
        out_ref[...] += (jnp.sum(jnp.log(sums) - diag) / bq).reshape(1, 1)

    out = pl.pallas_call(
        body,
        grid=(nqb,),
        in_specs=[
            pl.BlockSpec((qblk, DIMS), lambda i: (i, 0)),
            pl.BlockSpec((bd, DIMS), lambda i: (0, 0)),
            pl.BlockSpec((qblk, LPAD), lambda i: (i, 0)),
            pl.BlockSpec((bd, LPAD), lambda i: (0, 0)),
            pl.BlockSpec((1, DIMS), lambda i: (0, 0)),
            pl.BlockSpec((h, DIMS), lambda i: (0, 0)),
            pl.BlockSpec((1, h), lambda i: (0, 0)),
            pl.BlockSpec((h, DIMS), lambda i: (0, 0)),
            pl.BlockSpec((1, h), lambda i: (0, 0)),
        ],
        out_specs=pl.BlockSpec((1, 1), lambda i: (0, 0)),
        out_shape=jax.ShapeDtypeStruct((1, 1), jnp.float32),
        scratch_shapes=[pltpu.VMEM((bd, h), jnp.float32)],
    )(sum_q, sum_d, idx_q, idx_d, emb0, qw, qb.reshape(1, h), dw,
      db.reshape(1, h))
    return out[0, 0]


def kernel(query, doc, negs, emb, qd1_w, qd1_b, dd1_w, dd1_b):
    b = query.shape[0]
    idx = jnp.concatenate([query, doc, negs], axis=0)
    nrows = idx.shape[0]
    idxp = jnp.pad(idx, ((0, 0), (0, LPAD - L)))
    grp = jnp.pad(idx.reshape(nrows // GB, GW), ((0, 0), (0, GWP - GW)))
    ng = nrows // GB
    spread = (lax.broadcasted_iota(jnp.int32, (ng, GWP), 0) * 997
              + lax.broadcasted_iota(jnp.int32, (ng, GWP), 1) * 131) % NUMS_M
    col = lax.broadcasted_iota(jnp.int32, (ng, GWP), 1)
    grp = jnp.where(col < GW, grp, spread)
    gidx = grp.reshape(-1)
    embp = jnp.pad(emb, ((0, 0), (0, EMBW - DIMS)))
    sums = _sc_bag_sum(gidx, embp, nrows).reshape(nrows, DIMS)
    return _tc_head(sums[:b], sums[b:], idxp[:b], idxp[b:], emb[0:1],
                    qd1_w, qd1_b, dd1_w, dd1_b)
